# baseline (speedup 1.0000x reference)
import functools
import jax
import jax.numpy as jnp
from jax import lax
from jax.experimental import pallas as pl
from jax.experimental.pallas import tpu as pltpu
from jax.experimental.pallas import tpu_sc as plsc

E = 320000
D = 128
G = 256

E_SC = 223744
E_TC = E - E_SC

NC = 2
NS = 16
NW = NC * NS
NG = D // 16
CH = 128
NCH_ALL = E_SC // CH
NCH_BASE = NCH_ALL // NW
NCH_REM = NCH_ALL % NW
NCH_MAX = NCH_BASE + 1
NBUF = 4
LOOKAHEAD = 3
NQUAD = (NCH_MAX + NBUF - 1) // NBUF
RUNROLL = 16

EB = 512
NB_TC = E_TC // EB
KB0 = E_SC // EB


def _sc_body(feat_hbm, seg2_hbm, fidx_hbm, out_hbm,
             idx_v, fidx_v, acc_l,
             f0, f1, f2, f3, acc_sh,
             l0, l1, l2, l3, fsem):
    c = lax.axis_index("c")
    s = lax.axis_index("s")
    wid = s * NC + c
    chunk0 = wid * NCH_BASE + jnp.maximum(0, wid - (NW - NCH_REM))
    nchunk = NCH_BASE + jnp.where(wid >= NW - NCH_REM, 1, 0)
    base = chunk0 * CH
    bufs = (f0, f1, f2, f3)
    lsem = (l0, l1, l2, l3)

    zero = jnp.zeros((16,), jnp.float32)

    def zrow(r, carry):
        for g in range(NG):
            acc_l[r, pl.ds(g * 16, 16)] = zero
        return carry

    lax.fori_loop(0, G, zrow, 0)
    pltpu.sync_copy(acc_l.at[pl.ds(0, 16)], acc_sh.at[pl.ds(s * 16, 16)])

    pltpu.sync_copy(seg2_hbm.at[pl.ds(chunk0, NCH_MAX)], idx_v)
    pltpu.sync_copy(fidx_hbm, fidx_v)

    def feat_src(chunk):
        return feat_hbm.at[pl.ds(base + chunk * CH, CH)]

    for b in range(NBUF):
        pltpu.async_copy(feat_src(b), bufs[b], lsem[b])
    plsc.subcore_barrier()

    def quad_step(i, carry):
        for b in range(NBUF):
            chunk = NBUF * i + b
            buf = bufs[b]

            @pl.when(chunk < nchunk)
            def _():
                pltpu.make_async_copy(feat_src(chunk), buf, lsem[b]).wait()

                t = chunk + LOOKAHEAD
                bt = (b + LOOKAHEAD) % NBUF

                @pl.when((t >= NBUF) & (t < nchunk))
                def _():
                    pltpu.async_copy(feat_src(t), bufs[bt], lsem[bt])

                mn = jnp.min(idx_v[chunk, pl.ds(0, 16)])
                mx = jnp.max(idx_v[chunk, pl.ds(CH - 16, 16)])

                @pl.when(mx == mn)
                def _():
                    def srow(j, acc):
                        accs = list(acc)
                        for r in range(RUNROLL):
                            row = j * RUNROLL + r
                            for g in range(NG):
                                accs[g] = accs[g] + buf[row,
                                                        pl.ds(g * 16, 16)]
                        return tuple(accs)

                    sums = lax.fori_loop(0, CH // RUNROLL, srow,
                                         tuple(zero for _ in range(NG)))
                    for g in range(NG):
                        plsc.addupdate(acc_l.at[mx, pl.ds(g * 16, 16)],
                                       sums[g])

                @pl.when(mx != mn)
                def _():
                    pltpu.sync_copy(buf, acc_sh.at[idx_v.at[chunk]], add=True)

        return carry

    lax.fori_loop(0, NQUAD, quad_step, 0)

    pltpu.async_copy(acc_l.at[pl.ds(0, 128)], acc_sh.at[fidx_v.at[0]], fsem,
                     add=True)
    pltpu.async_copy(acc_l.at[pl.ds(128, 128)], acc_sh.at[fidx_v.at[1]], fsem,
                     add=True)
    pltpu.make_async_copy(acc_l.at[pl.ds(0, 128)], acc_sh.at[fidx_v.at[0]],
                          fsem).wait()
    pltpu.make_async_copy(acc_l.at[pl.ds(128, 128)], acc_sh.at[fidx_v.at[1]],
                          fsem).wait()
    plsc.subcore_barrier()

    pltpu.sync_copy(acc_sh.at[pl.ds(s * 16, 16)],
                    out_hbm.at[c, pl.ds(s * 16, 16)])


def _sc_partials(feat, seg2, fidx):
    mesh = plsc.VectorSubcoreMesh(core_axis_name="c", subcore_axis_name="s")
    f = pl.kernel(
        _sc_body,
        out_type=jax.ShapeDtypeStruct((NC, G, D), jnp.float32),
        mesh=mesh,
        scratch_types=(
            [pltpu.VMEM((NCH_MAX, CH), jnp.int32),
             pltpu.VMEM((2, 128), jnp.int32),
             pltpu.VMEM((G, D), jnp.float32)]
            + [pltpu.VMEM((CH, D), jnp.float32)] * NBUF
            + [pltpu.VMEM_SHARED((G, D), jnp.float32)]
            + [pltpu.SemaphoreType.DMA] * (NBUF + 1)
        ),
        compiler_params=pltpu.CompilerParams(use_tc_tiling_on_sc=False,
                                             needs_layout_passes=False),
        name="segment_sum_pool_sc",
    )
    return f(feat, seg2, fidx)


def _tc_block(seg_ref, feat_ref, out_ref):
    k = pl.program_id(0)

    @pl.when(k == 0)
    def _():
        out_ref[...] = jnp.zeros((G, D), jnp.float32)

    seg = seg_ref[0, 0, :]
    onehot = (lax.broadcasted_iota(jnp.int32, (G, EB), 0)
              == seg[None, :]).astype(jnp.bfloat16)
    out_ref[...] += jnp.dot(onehot, feat_ref[...].astype(jnp.bfloat16),
                            preferred_element_type=jnp.float32)


def _tc_partial(feat, seg3):
    return pl.pallas_call(
        _tc_block,
        grid=(NB_TC,),
        in_specs=[
            pl.BlockSpec((1, 1, EB), lambda k: (KB0 + k, 0, 0)),
            pl.BlockSpec((EB, D), lambda k: (KB0 + k, 0)),
        ],
        out_specs=pl.BlockSpec((G, D), lambda k: (0, 0)),
        out_shape=jax.ShapeDtypeStruct((G, D), jnp.float32),
        name="segment_sum_pool_tc",
    )(seg3, feat)


def _add_block(p_ref, t_ref, o_ref):
    o_ref[...] = p_ref[0] + p_ref[1] + t_ref[...]


def _combine(p, t):
    return pl.pallas_call(
        _add_block,
        out_shape=jax.ShapeDtypeStruct((G, D), jnp.float32),
        name="segment_sum_combine",
    )(p, t)


@jax.jit
def _sum_pool(feat, segment_ids):
    fidx = jnp.arange(G, dtype=jnp.int32).reshape(2, 128)
    tc = _tc_partial(feat, segment_ids.reshape(E // EB, 1, EB))
    partials = _sc_partials(feat, segment_ids.reshape(E // CH, CH), fidx)
    return _combine(partials, tc)


def kernel(feat, segment_ids, num_graphs):
    num_graphs = jnp.asarray(num_graphs, dtype=jnp.int32)
    segment_ids = segment_ids + (num_graphs - jnp.int32(G))
    return _sum_pool(feat, segment_ids)


# speedup vs baseline: 1.7362x; 1.7362x over previous
import functools
import jax
import jax.numpy as jnp
from jax import lax
from jax.experimental import pallas as pl
from jax.experimental.pallas import tpu as pltpu
from jax.experimental.pallas import tpu_sc as plsc

E = 320000
D = 128
G = 256

NC = 2
NS = 16
NW = NC * NS
NG = D // 16
CH = 128
NCH_ALL = E // CH
NCH_BASE = NCH_ALL // NW
NCH_REM = NCH_ALL % NW
NCH_MAX = NCH_BASE + 1
NBUF = 4
LOOKAHEAD = 3
NQUAD = (NCH_MAX + NBUF - 1) // NBUF
RUNROLL = 16


def _sc_body(feat_hbm, seg2_hbm, fidx_hbm, out_hbm,
             idx_v, fidx_v, acc_l,
             f0, f1, f2, f3, acc_sh,
             l0, l1, l2, l3, fsem):
    c = lax.axis_index("c")
    s = lax.axis_index("s")
    wid = s * NC + c
    chunk0 = wid * NCH_BASE + jnp.maximum(0, wid - (NW - NCH_REM))
    nchunk = NCH_BASE + jnp.where(wid >= NW - NCH_REM, 1, 0)
    base = chunk0 * CH
    bufs = (f0, f1, f2, f3)
    lsem = (l0, l1, l2, l3)

    zero = jnp.zeros((16,), jnp.float32)

    def zrow(r, carry):
        for g in range(NG):
            acc_l[r, pl.ds(g * 16, 16)] = zero
        return carry

    lax.fori_loop(0, G, zrow, 0)
    pltpu.sync_copy(acc_l.at[pl.ds(0, 16)], acc_sh.at[pl.ds(s * 16, 16)])

    pltpu.sync_copy(seg2_hbm.at[pl.ds(chunk0, NCH_MAX)], idx_v)
    pltpu.sync_copy(fidx_hbm, fidx_v)

    def feat_src(chunk):
        return feat_hbm.at[pl.ds(base + chunk * CH, CH)]

    for b in range(NBUF):
        pltpu.async_copy(feat_src(b), bufs[b], lsem[b])
    plsc.subcore_barrier()

    def quad_step(i, carry):
        for b in range(NBUF):
            chunk = NBUF * i + b
            buf = bufs[b]

            @pl.when(chunk < nchunk)
            def _():
                pltpu.make_async_copy(feat_src(chunk), buf, lsem[b]).wait()

                t = chunk + LOOKAHEAD
                bt = (b + LOOKAHEAD) % NBUF

                @pl.when((t >= NBUF) & (t < nchunk))
                def _():
                    pltpu.async_copy(feat_src(t), bufs[bt], lsem[bt])

                mn = jnp.min(idx_v[chunk, pl.ds(0, 16)])
                mx = jnp.max(idx_v[chunk, pl.ds(CH - 16, 16)])

                @pl.when(mx == mn)
                def _():
                    def srow(j, acc):
                        accs = list(acc)
                        for r in range(RUNROLL):
                            row = j * RUNROLL + r
                            for g in range(NG):
                                accs[g] = accs[g] + buf[row,
                                                        pl.ds(g * 16, 16)]
                        return tuple(accs)

                    sums = lax.fori_loop(0, CH // RUNROLL, srow,
                                         tuple(zero for _ in range(NG)))
                    for g in range(NG):
                        plsc.addupdate(acc_l.at[mx, pl.ds(g * 16, 16)],
                                       sums[g])

                @pl.when(mx != mn)
                def _():
                    pltpu.sync_copy(buf, acc_sh.at[idx_v.at[chunk]], add=True)

        return carry

    lax.fori_loop(0, NQUAD, quad_step, 0)

    pltpu.async_copy(acc_l.at[pl.ds(0, 128)], acc_sh.at[fidx_v.at[0]], fsem,
                     add=True)
    pltpu.async_copy(acc_l.at[pl.ds(128, 128)], acc_sh.at[fidx_v.at[1]], fsem,
                     add=True)
    pltpu.make_async_copy(acc_l.at[pl.ds(0, 128)], acc_sh.at[fidx_v.at[0]],
                          fsem).wait()
    pltpu.make_async_copy(acc_l.at[pl.ds(128, 128)], acc_sh.at[fidx_v.at[1]],
                          fsem).wait()
    plsc.subcore_barrier()

    pltpu.sync_copy(acc_sh.at[pl.ds(s * 16, 16)],
                    out_hbm.at[c, pl.ds(s * 16, 16)])


def _sc_partials(feat, seg2, fidx):
    mesh = plsc.VectorSubcoreMesh(core_axis_name="c", subcore_axis_name="s")
    f = pl.kernel(
        _sc_body,
        out_type=jax.ShapeDtypeStruct((NC, G, D), jnp.float32),
        mesh=mesh,
        scratch_types=(
            [pltpu.VMEM((NCH_MAX, CH), jnp.int32),
             pltpu.VMEM((2, 128), jnp.int32),
             pltpu.VMEM((G, D), jnp.float32)]
            + [pltpu.VMEM((CH, D), jnp.float32)] * NBUF
            + [pltpu.VMEM_SHARED((G, D), jnp.float32)]
            + [pltpu.SemaphoreType.DMA] * (NBUF + 1)
        ),
        compiler_params=pltpu.CompilerParams(use_tc_tiling_on_sc=False,
                                             needs_layout_passes=False),
        name="segment_sum_pool_sc",
    )
    return f(feat, seg2, fidx)


def _add_block(p_ref, o_ref):
    o_ref[...] = p_ref[0] + p_ref[1]


def _combine(p):
    return pl.pallas_call(
        _add_block,
        out_shape=jax.ShapeDtypeStruct((G, D), jnp.float32),
        name="segment_sum_combine",
    )(p)


@jax.jit
def _sum_pool(feat, segment_ids):
    fidx = jnp.arange(G, dtype=jnp.int32).reshape(2, 128)
    partials = _sc_partials(feat, segment_ids.reshape(E // CH, CH), fidx)
    return _combine(partials)


def kernel(feat, segment_ids, num_graphs):
    num_graphs = jnp.asarray(num_graphs, dtype=jnp.int32)
    segment_ids = segment_ids + (num_graphs - jnp.int32(G))
    return _sum_pool(feat, segment_ids)
